# Initial kernel scaffold; baseline (speedup 1.0000x reference)
import jax
import jax.numpy as jnp
from jax.experimental import pallas as pl


def kernel(x, conv_w, conv_b, fc1_w, fc1_b, fc2_w, fc2_b):
    raise NotImplementedError("write your pallas kernel here")


# speedup vs baseline: 4.7943x; 4.7943x over previous
import functools

import jax
import jax.numpy as jnp
from jax.experimental import pallas as pl
from jax.experimental.pallas import tpu as pltpu

BT = 128
PH = 384
NCOL = 2 * PH - 124


def _fused_kernel(x_ref, ws_ref, bc_ref, w1_ref, b1_ref, w2_ref, b2_ref,
                  o_ref, xp_ref):
    bt = x_ref.shape[0]
    xp_ref[...] = jnp.zeros((bt, 30, 30), jnp.float32)
    xp_ref[:, 1:29, 1:29] = x_ref[...]

    strips = jnp.concatenate(
        [xp_ref[:, ky:ky + 26, :] for ky in range(5)], axis=2)
    s2 = strips.reshape(bt * 26, 150)

    conv = jnp.dot(s2, ws_ref[...], preferred_element_type=jnp.float32)
    conv = jnp.maximum(conv + bc_ref[...], 0.0)
    c3 = conv.reshape(bt, 26, NCOL)

    m = jnp.maximum(c3[:, :, 0:260], c3[:, :, PH:PH + 260])

    h = b1_ref[...]
    for i in range(13):
        pi = jnp.max(m[:, 2 * i:2 * i + 2, :], axis=1)
        h = h + jnp.dot(pi, w1_ref[i], preferred_element_type=jnp.float32)
    h = jnp.maximum(h, 0.0)

    o_ref[...] = (jnp.dot(h, w2_ref[...], preferred_element_type=jnp.float32)
                  + b2_ref[...])


@jax.jit
def _forward(x, conv_w, conv_b, fc1_w, fc1_b, fc2_w, fc2_b):
    B = x.shape[0]
    x3 = x.reshape(B, 28, 28)
    Bp = -(-B // BT) * BT
    if Bp != B:
        x3 = jnp.pad(x3, ((0, Bp - B), (0, 0), (0, 0)))

    wc = conv_w.reshape(20, 5, 5)
    c = jnp.arange(20)[:, None, None, None, None]
    j = jnp.arange(13)[None, :, None, None, None]
    ph = jnp.arange(2)[None, None, :, None, None]
    ky = jnp.arange(5)[None, None, None, :, None]
    kx = jnp.arange(5)[None, None, None, None, :]
    full = (20, 13, 2, 5, 5)
    rows = jnp.broadcast_to(ky * 30 + 2 * j + ph + kx, full)
    cols = jnp.broadcast_to(ph * PH + c * 13 + j, full)
    vals = jnp.broadcast_to(wc[c, ky, kx], full)
    ws = jnp.zeros((150, NCOL), jnp.float32).at[
        rows.reshape(-1), cols.reshape(-1)].set(vals.reshape(-1))

    bcol = jnp.zeros((1, NCOL), jnp.float32).at[
        0, jnp.broadcast_to(ph * PH + c * 13 + j,
                            (20, 13, 2, 1, 1)).reshape(-1)
    ].set(jnp.broadcast_to(conv_b[:, None, None, None, None],
                           (20, 13, 2, 1, 1)).reshape(-1))

    w1r = fc1_w.reshape(100, 20, 13, 13).transpose(2, 1, 3, 0)
    w1r = w1r.reshape(13, 260, 100)
    b1 = fc1_b.reshape(1, 100)
    w2 = fc2_w.T
    b2 = fc2_b.reshape(1, 10)

    out = pl.pallas_call(
        _fused_kernel,
        out_shape=jax.ShapeDtypeStruct((Bp, 10), jnp.float32),
        grid=(Bp // BT,),
        in_specs=[
            pl.BlockSpec((BT, 28, 28), lambda i: (i, 0, 0)),
            pl.BlockSpec((150, NCOL), lambda i: (0, 0)),
            pl.BlockSpec((1, NCOL), lambda i: (0, 0)),
            pl.BlockSpec((13, 260, 100), lambda i: (0, 0, 0)),
            pl.BlockSpec((1, 100), lambda i: (0, 0)),
            pl.BlockSpec((100, 10), lambda i: (0, 0)),
            pl.BlockSpec((1, 10), lambda i: (0, 0)),
        ],
        out_specs=pl.BlockSpec((BT, 10), lambda i: (i, 0)),
        scratch_shapes=[pltpu.VMEM((BT, 30, 30), jnp.float32)],
        compiler_params=pltpu.CompilerParams(
            dimension_semantics=("parallel",)),
    )(x3, ws, bcol, w1r, b1, w2, b2)
    return out[:B]


def kernel(x, conv_w, conv_b, fc1_w, fc1_b, fc2_w, fc2_b):
    return _forward(x, conv_w, conv_b, fc1_w, fc1_b, fc2_w, fc2_b)


# speedup vs baseline: 10.0872x; 2.1040x over previous
import jax
import jax.numpy as jnp
from jax.experimental import pallas as pl
from jax.experimental.pallas import tpu as pltpu

BT = 128
PH = 384
NCOL = 2 * PH - 124


def _fused_kernel(xd_ref, ws_ref, bc_ref, w1_ref, b1_ref, w2_ref, b2_ref,
                  o_ref, pl0_ref, pl1_ref, ms_ref):
    bt = xd_ref.shape[0]
    pl0_ref[...] = jnp.zeros((bt, 24, 30), jnp.float32)
    pl1_ref[...] = jnp.zeros((bt, 24, 30), jnp.float32)
    pl0_ref[:, 1:15, 1:29] = xd_ref[:, 1]
    pl1_ref[:, 0:14, 1:29] = xd_ref[:, 0]

    se = jnp.concatenate(
        [pl0_ref[:, 0:16, :], pl1_ref[:, 0:16, :], pl0_ref[:, 1:17, :],
         pl1_ref[:, 1:17, :], pl0_ref[:, 2:18, :]], axis=2)
    so = jnp.concatenate(
        [pl1_ref[:, 0:16, :], pl0_ref[:, 1:17, :], pl1_ref[:, 1:17, :],
         pl0_ref[:, 2:18, :], pl1_ref[:, 2:18, :]], axis=2)

    ws = ws_ref[...]
    ce = jnp.dot(se.reshape(bt * 16, 150), ws,
                 preferred_element_type=jnp.float32)
    co = jnp.dot(so.reshape(bt * 16, 150), ws,
                 preferred_element_type=jnp.float32)

    t = jnp.maximum(ce, co)
    m = jnp.maximum(t[:, 0:260], t[:, PH:PH + 260])
    m = jnp.maximum(m + bc_ref[...], 0.0)

    ms_ref[...] = m.reshape(bt, 16, 260)
    h = b1_ref[...]
    for i in range(13):
        h = h + jnp.dot(ms_ref[:, i, :], w1_ref[i],
                        preferred_element_type=jnp.float32)
    h = jnp.maximum(h, 0.0)
    o_ref[...] = (jnp.dot(h, w2_ref[...], preferred_element_type=jnp.float32)
                  + b2_ref[...])


@jax.jit
def _forward(x, conv_w, conv_b, fc1_w, fc1_b, fc2_w, fc2_b):
    B = x.shape[0]
    x3 = x.reshape(B, 28, 28)
    Bp = -(-B // BT) * BT
    if Bp != B:
        x3 = jnp.pad(x3, ((0, Bp - B), (0, 0), (0, 0)))
    xd = x3.reshape(Bp, 14, 2, 28).transpose(0, 2, 1, 3)

    wc = conv_w.reshape(20, 5, 5)
    c = jnp.arange(20)[:, None, None, None, None]
    j = jnp.arange(13)[None, :, None, None, None]
    ph = jnp.arange(2)[None, None, :, None, None]
    ky = jnp.arange(5)[None, None, None, :, None]
    kx = jnp.arange(5)[None, None, None, None, :]
    full = (20, 13, 2, 5, 5)
    rows = jnp.broadcast_to(ky * 30 + 2 * j + ph + kx, full)
    cols = jnp.broadcast_to(ph * PH + c * 13 + j, full)
    vals = jnp.broadcast_to(wc[c, ky, kx], full)
    ws = jnp.zeros((150, NCOL), jnp.float32).at[
        rows.reshape(-1), cols.reshape(-1)].set(vals.reshape(-1))

    bc = jnp.broadcast_to(conv_b[:, None], (20, 13)).reshape(1, 260)

    w1r = fc1_w.reshape(100, 20, 13, 13).transpose(2, 1, 3, 0)
    w1r = w1r.reshape(13, 260, 100)
    b1 = fc1_b.reshape(1, 100)
    w2 = fc2_w.T
    b2 = fc2_b.reshape(1, 10)

    out = pl.pallas_call(
        _fused_kernel,
        out_shape=jax.ShapeDtypeStruct((Bp, 10), jnp.float32),
        grid=(Bp // BT,),
        in_specs=[
            pl.BlockSpec((BT, 2, 14, 28), lambda i: (i, 0, 0, 0)),
            pl.BlockSpec((150, NCOL), lambda i: (0, 0)),
            pl.BlockSpec((1, 260), lambda i: (0, 0)),
            pl.BlockSpec((13, 260, 100), lambda i: (0, 0, 0)),
            pl.BlockSpec((1, 100), lambda i: (0, 0)),
            pl.BlockSpec((100, 10), lambda i: (0, 0)),
            pl.BlockSpec((1, 10), lambda i: (0, 0)),
        ],
        out_specs=pl.BlockSpec((BT, 10), lambda i: (i, 0)),
        scratch_shapes=[pltpu.VMEM((BT, 24, 30), jnp.float32),
                        pltpu.VMEM((BT, 24, 30), jnp.float32),
                        pltpu.VMEM((BT, 16, 260), jnp.float32)],
        compiler_params=pltpu.CompilerParams(
            dimension_semantics=("parallel",)),
    )(xd, ws, bc, w1r, b1, w2, b2)
    return out[:B]


def kernel(x, conv_w, conv_b, fc1_w, fc1_b, fc2_w, fc2_b):
    return _forward(x, conv_w, conv_b, fc1_w, fc1_b, fc2_w, fc2_b)


# speedup vs baseline: 12.4220x; 1.2315x over previous
import jax
import jax.numpy as jnp
from jax.experimental import pallas as pl
from jax.experimental.pallas import tpu as pltpu

BT = 256
PH = 384
NCOL = 2 * PH - 124


def _fused_kernel(xd_ref, ws_ref, bc_ref, w1_ref, b1_ref, w2_ref, b2_ref,
                  o_ref, pl0_ref, pl1_ref):
    bt = o_ref.shape[0]
    pl0_ref[...] = jnp.zeros(pl0_ref.shape, jnp.float32)
    pl1_ref[...] = jnp.zeros(pl1_ref.shape, jnp.float32)
    pl0_ref[1:15, :, 1:29] = xd_ref[1]
    pl1_ref[0:14, :, 1:29] = xd_ref[0]

    se = jnp.concatenate(
        [pl0_ref[0:16], pl1_ref[0:16], pl0_ref[1:17],
         pl1_ref[1:17], pl0_ref[2:18]], axis=2)
    so = jnp.concatenate(
        [pl1_ref[0:16], pl0_ref[1:17], pl1_ref[1:17],
         pl0_ref[2:18], pl1_ref[2:18]], axis=2)

    ws = ws_ref[...]
    ce = jnp.dot(se.reshape(16 * bt, 150), ws,
                 preferred_element_type=jnp.float32)
    co = jnp.dot(so.reshape(16 * bt, 150), ws,
                 preferred_element_type=jnp.float32)

    t = jnp.maximum(ce, co)
    m = jnp.maximum(t[:, 0:260], t[:, PH:PH + 260])
    m = jnp.maximum(m + bc_ref[...], 0.0)

    mi = m.reshape(16, bt, 260)
    h = b1_ref[...]
    for i in range(13):
        h = h + jnp.dot(mi[i], w1_ref[i],
                        preferred_element_type=jnp.float32)
    h = jnp.maximum(h, 0.0)
    o_ref[...] = (jnp.dot(h, w2_ref[...], preferred_element_type=jnp.float32)
                  + b2_ref[...])


@jax.jit
def _forward(x, conv_w, conv_b, fc1_w, fc1_b, fc2_w, fc2_b):
    B = x.shape[0]
    x3 = x.reshape(B, 28, 28)
    Bp = -(-B // BT) * BT
    if Bp != B:
        x3 = jnp.pad(x3, ((0, Bp - B), (0, 0), (0, 0)))
    xd = x3.reshape(Bp, 14, 2, 28).transpose(2, 1, 0, 3)

    wc = conv_w.reshape(20, 5, 5)
    c = jnp.arange(20)[:, None, None, None, None]
    j = jnp.arange(13)[None, :, None, None, None]
    ph = jnp.arange(2)[None, None, :, None, None]
    ky = jnp.arange(5)[None, None, None, :, None]
    kx = jnp.arange(5)[None, None, None, None, :]
    full = (20, 13, 2, 5, 5)
    rows = jnp.broadcast_to(ky * 30 + 2 * j + ph + kx, full)
    cols = jnp.broadcast_to(ph * PH + c * 13 + j, full)
    vals = jnp.broadcast_to(wc[c, ky, kx], full)
    ws = jnp.zeros((150, NCOL), jnp.float32).at[
        rows.reshape(-1), cols.reshape(-1)].set(vals.reshape(-1))

    bc = jnp.broadcast_to(conv_b[:, None], (20, 13)).reshape(1, 260)

    w1r = fc1_w.reshape(100, 20, 13, 13).transpose(2, 1, 3, 0)
    w1r = w1r.reshape(13, 260, 100)
    b1 = fc1_b.reshape(1, 100)
    w2 = fc2_w.T
    b2 = fc2_b.reshape(1, 10)

    out = pl.pallas_call(
        _fused_kernel,
        out_shape=jax.ShapeDtypeStruct((Bp, 10), jnp.float32),
        grid=(Bp // BT,),
        in_specs=[
            pl.BlockSpec((2, 14, BT, 28), lambda i: (0, 0, i, 0)),
            pl.BlockSpec((150, NCOL), lambda i: (0, 0)),
            pl.BlockSpec((1, 260), lambda i: (0, 0)),
            pl.BlockSpec((13, 260, 100), lambda i: (0, 0, 0)),
            pl.BlockSpec((1, 100), lambda i: (0, 0)),
            pl.BlockSpec((100, 10), lambda i: (0, 0)),
            pl.BlockSpec((1, 10), lambda i: (0, 0)),
        ],
        out_specs=pl.BlockSpec((BT, 10), lambda i: (i, 0)),
        scratch_shapes=[pltpu.VMEM((18, BT, 30), jnp.float32),
                        pltpu.VMEM((18, BT, 30), jnp.float32)],
        compiler_params=pltpu.CompilerParams(
            dimension_semantics=("parallel",)),
    )(xd, ws, bc, w1r, b1, w2, b2)
    return out[:B]


def kernel(x, conv_w, conv_b, fc1_w, fc1_b, fc2_w, fc2_b):
    return _forward(x, conv_w, conv_b, fc1_w, fc1_b, fc2_w, fc2_b)
